# baseline (speedup 1.0000x reference)
import functools

import jax
import jax.numpy as jnp
from jax import lax
from jax.experimental import pallas as pl
from jax.experimental.pallas import tpu as pltpu

H = 512
W = 512
WINDOW = 12

BLK = 2048


def _image_kernel(px_ref, py_ref, h_ref, w_ref, bg_ref, out_ref):
    k = pl.program_id(0)

    px = px_ref[...]
    py = py_ref[...]
    height = h_ref[...]
    width = w_ref[...]
    s = jnp.sqrt(0.5 * 1.4426950408889634) / width

    cols = lax.broadcasted_iota(jnp.int32, (BLK, W), 1).astype(jnp.float32)
    dx = cols * s[:, None] - (px * s)[:, None]
    fx = jnp.exp2(-(dx * dx))
    dy = cols * s[:, None] - (py * s)[:, None]
    fy = height[:, None] * jnp.exp2(-(dy * dy))

    acc = lax.dot_general(
        fy.astype(jnp.bfloat16), fx.astype(jnp.bfloat16),
        (((0,), (0,)), ((), ())),
        preferred_element_type=jnp.float32,
    )

    @pl.when(k == 0)
    def _():
        out_ref[...] = jnp.full((H, W), bg_ref[0, 0], jnp.float32)

    out_ref[...] += acc


def _kernel_tc(pos_x, pos_y, height, width, background):
    n = pos_x.shape[0]
    n_pad = ((n + BLK - 1) // BLK) * BLK
    pad = n_pad - n
    pos_x = jnp.pad(pos_x, (0, pad))
    pos_y = jnp.pad(pos_y, (0, pad))
    height = jnp.pad(height, (0, pad))
    width = jnp.pad(width, (0, pad), constant_values=1.0)
    bg = jnp.reshape(background, (1, 1)).astype(jnp.float32)

    grid = n_pad // BLK
    peaks_spec = pl.BlockSpec((BLK,), lambda k: (k,))
    return pl.pallas_call(
        _image_kernel,
        grid=(grid,),
        in_specs=[peaks_spec, peaks_spec, peaks_spec, peaks_spec,
                  pl.BlockSpec(memory_space=pltpu.SMEM)],
        out_specs=pl.BlockSpec((H, W), lambda k: (0, 0)),
        out_shape=jax.ShapeDtypeStruct((H, W), jnp.float32),
    )(pos_x, pos_y, height, width, bg)


from jax.experimental.pallas import tpu_sc as plsc

BAND = 16
NW = 32
NPAD = 10240
ACC_W = W + 2 * BAND
DWIN = 2 * WINDOW + 1


def _sc_body(px_hbm, py_hbm, h_hbm, w_hbm, bg_hbm, out_hbm,
             px_v, py_v, h_v, w_v, bg_v, wl_v, hy_v, acc_v, stage_v):
    wid = lax.axis_index("s") * 2 + lax.axis_index("c")
    base_row = wid * BAND

    pltpu.sync_copy(px_hbm, px_v)
    pltpu.sync_copy(py_hbm, py_v)
    pltpu.sync_copy(h_hbm, h_v)
    pltpu.sync_copy(w_hbm, w_v)
    pltpu.sync_copy(bg_hbm, bg_v)

    iota = lax.iota(jnp.int32, 16)
    bgv = bg_v[...]

    dummy = jnp.full((16,), NPAD - 1, jnp.int32)

    def _wl_init(i, _):
        wl_v[pl.ds(i * 16, 16)] = dummy
        return 0

    lax.fori_loop(0, (NPAD + 16) // 16, _wl_init, 0)

    def _acc_init(i, _):
        acc_v[pl.ds(i * 16, 16)] = bgv
        return 0

    lax.fori_loop(0, (BAND * ACC_W) // 16, _acc_init, 0)

    lo = base_row - WINDOW
    hi = base_row + BAND - 1 + WINDOW

    def _scan(i, total):
        py16 = py_v[pl.ds(i * 16, 16)]
        ry = py16.astype(jnp.int32)
        m = (ry >= lo) & (ry <= hi)
        offs = total + plsc.cumsum(m.astype(jnp.int32)) - 1
        plsc.store_scatter(wl_v, [offs], iota + i * 16, mask=m)
        return total + plsc.all_reduce_population_count(m)

    zero_v = jnp.zeros((16,), jnp.int32)
    total_v = lax.fori_loop(0, NPAD // 16, _scan, zero_v)
    total = total_v[0]

    def _chunk(t, _):
        idx = wl_v[pl.ds(t * 16, 16)]
        pxv = plsc.load_gather(px_v, [idx])
        pyv = plsc.load_gather(py_v, [idx])
        hv = plsc.load_gather(h_v, [idx])
        wv = plsc.load_gather(w_v, [idx])
        ninv = -0.5 / (wv * wv)
        pxi = pxv.astype(jnp.int32)
        fx = pxv - pxi.astype(jnp.float32)
        colbase = pxi + (BAND - WINDOW)

        hx = []
        for d in range(DWIN):
            dd = (d - WINDOW) - fx
            hx.append(jnp.exp((dd * dd) * ninv))

        for r in range(BAND):
            dr = (base_row + r) - pyv
            hy_v[pl.ds(r * 16, 16)] = hv * jnp.exp((dr * dr) * ninv)

        for e in range(BAND):
            rot = (iota + e) & 15
            hy_e = plsc.load_gather(hy_v, [rot * 16 + iota])
            base_addr = rot * ACC_W + colbase
            for d in range(DWIN):
                plsc.addupdate_scatter(acc_v, [base_addr + d], hy_e * hx[d])
        return 0

    lax.fori_loop(0, (total + 15) // 16, _chunk, 0)

    for r in range(BAND):
        for c in range(W // 16):
            stage_v[r, pl.ds(c * 16, 16)] = (
                acc_v[pl.ds(r * ACC_W + BAND + c * 16, 16)])
    pltpu.sync_copy(stage_v, out_hbm.at[pl.ds(base_row, BAND)])


_sc_call = functools.partial(
    pl.kernel,
    mesh=plsc.VectorSubcoreMesh(core_axis_name="c", subcore_axis_name="s"),
    out_type=jax.ShapeDtypeStruct((H, W), jnp.float32),
    compiler_params=pltpu.CompilerParams(needs_layout_passes=False),
    scratch_types=[
        pltpu.VMEM((NPAD,), jnp.float32),
        pltpu.VMEM((NPAD,), jnp.float32),
        pltpu.VMEM((NPAD,), jnp.float32),
        pltpu.VMEM((NPAD,), jnp.float32),
        pltpu.VMEM((16,), jnp.float32),
        pltpu.VMEM((NPAD + 16,), jnp.int32),
        pltpu.VMEM((BAND * 16,), jnp.float32),
        pltpu.VMEM((BAND * ACC_W,), jnp.float32),
        pltpu.VMEM((BAND, W), jnp.float32),
    ],
)(_sc_body)


def _kernel_sc(pos_x, pos_y, height, width, background):
    n = pos_x.shape[0]
    pad = NPAD - n
    px = jnp.pad(pos_x, (0, pad))
    py = jnp.pad(pos_y, (0, pad), constant_values=1e6)
    h = jnp.pad(height, (0, pad))
    w = jnp.pad(width, (0, pad), constant_values=1.0)
    bg = jnp.full((16,), background, jnp.float32)
    return _sc_call(px, py, h, w, bg)


def kernel(x_grid, y_grid, pos_x, pos_y, height, width, background):
    return _kernel_sc(pos_x, pos_y, height, width, background)


# speedup vs baseline: 1.8529x; 1.8529x over previous
import functools

import jax
import jax.numpy as jnp
from jax import lax
from jax.experimental import pallas as pl
from jax.experimental.pallas import tpu as pltpu

H = 512
W = 512
WINDOW = 12

BLK = 2048


def _image_kernel(px_ref, py_ref, h_ref, w_ref, bg_ref, out_ref):
    k = pl.program_id(0)

    px = px_ref[...]
    py = py_ref[...]
    height = h_ref[...]
    width = w_ref[...]
    s = jnp.sqrt(0.5 * 1.4426950408889634) / width

    cols = lax.broadcasted_iota(jnp.int32, (BLK, W), 1).astype(jnp.float32)
    dx = cols * s[:, None] - (px * s)[:, None]
    fx = jnp.exp2(-(dx * dx))
    dy = cols * s[:, None] - (py * s)[:, None]
    fy = height[:, None] * jnp.exp2(-(dy * dy))

    acc = lax.dot_general(
        fy.astype(jnp.bfloat16), fx.astype(jnp.bfloat16),
        (((0,), (0,)), ((), ())),
        preferred_element_type=jnp.float32,
    )

    @pl.when(k == 0)
    def _():
        out_ref[...] = jnp.full((H, W), bg_ref[0, 0], jnp.float32)

    out_ref[...] += acc


def _kernel_tc(pos_x, pos_y, height, width, background):
    n = pos_x.shape[0]
    n_pad = ((n + BLK - 1) // BLK) * BLK
    pad = n_pad - n
    pos_x = jnp.pad(pos_x, (0, pad))
    pos_y = jnp.pad(pos_y, (0, pad))
    height = jnp.pad(height, (0, pad))
    width = jnp.pad(width, (0, pad), constant_values=1.0)
    bg = jnp.reshape(background, (1, 1)).astype(jnp.float32)

    grid = n_pad // BLK
    peaks_spec = pl.BlockSpec((BLK,), lambda k: (k,))
    return pl.pallas_call(
        _image_kernel,
        grid=(grid,),
        in_specs=[peaks_spec, peaks_spec, peaks_spec, peaks_spec,
                  pl.BlockSpec(memory_space=pltpu.SMEM)],
        out_specs=pl.BlockSpec((H, W), lambda k: (0, 0)),
        out_shape=jax.ShapeDtypeStruct((H, W), jnp.float32),
    )(pos_x, pos_y, height, width, bg)


from jax.experimental.pallas import tpu_sc as plsc

BAND = 16
NW = 32
NPAD = 10240
ACC_W = W + 2 * BAND
DWIN = 2 * WINDOW + 1


def _sc_body(npad, px_hbm, py_hbm, h_hbm, w_hbm, bg_hbm, out_hbm,
             px_v, py_v, h_v, w_v, bg_v, wl_v, hy_v, acc_v, stage_v):
    wid = lax.axis_index("s") * 2 + lax.axis_index("c")
    base_row = wid * BAND

    pltpu.sync_copy(px_hbm, px_v)
    pltpu.sync_copy(py_hbm, py_v)
    pltpu.sync_copy(h_hbm, h_v)
    pltpu.sync_copy(w_hbm, w_v)
    pltpu.sync_copy(bg_hbm, bg_v)

    iota = lax.iota(jnp.int32, 16)
    bgv = bg_v[...]

    dummy = jnp.full((16,), npad - 1, jnp.int32)

    def _wl_init(i, _):
        wl_v[pl.ds(i * 16, 16)] = dummy
        return 0

    lax.fori_loop(0, (npad + 16) // 16, _wl_init, 0)

    def _acc_init(i, _):
        acc_v[pl.ds(i * 16, 16)] = bgv
        return 0

    lax.fori_loop(0, (BAND * ACC_W) // 16, _acc_init, 0)

    lo = base_row - WINDOW
    hi = base_row + BAND - 1 + WINDOW

    def _scan(i, total):
        py16 = py_v[pl.ds(i * 16, 16)]
        ry = py16.astype(jnp.int32)
        m = (ry >= lo) & (ry <= hi)
        offs = total + plsc.cumsum(m.astype(jnp.int32)) - 1
        plsc.store_scatter(wl_v, [offs], iota + i * 16, mask=m)
        return total + plsc.all_reduce_population_count(m)

    zero_v = jnp.zeros((16,), jnp.int32)
    total_v = lax.fori_loop(0, npad // 16, _scan, zero_v)
    total = total_v[0]

    def _chunk(t, _):
        idx = wl_v[pl.ds(t * 16, 16)]
        pxv = plsc.load_gather(px_v, [idx])
        pyv = plsc.load_gather(py_v, [idx])
        hv = plsc.load_gather(h_v, [idx])
        wv = plsc.load_gather(w_v, [idx])
        ninv = -0.5 / (wv * wv)
        pxi = pxv.astype(jnp.int32)
        fx = pxv - pxi.astype(jnp.float32)
        colbase = pxi + (BAND - WINDOW)

        hx = []
        for d in range(DWIN):
            dd = (d - WINDOW) - fx
            hx.append(jnp.exp((dd * dd) * ninv))

        for r in range(BAND):
            dr = (base_row + r) - pyv
            hy_v[pl.ds(r * 16, 16)] = hv * jnp.exp((dr * dr) * ninv)

        for e in range(BAND):
            rot = (iota + e) & 15
            hy_e = plsc.load_gather(hy_v, [rot * 16 + iota])
            base_addr = rot * ACC_W + colbase
            for d in range(DWIN):
                plsc.addupdate_scatter(acc_v, [base_addr + d], hy_e * hx[d])
        return 0

    lax.fori_loop(0, (total + 15) // 16, _chunk, 0)

    for r in range(BAND):
        for c in range(W // 16):
            stage_v[r, pl.ds(c * 16, 16)] = (
                acc_v[pl.ds(r * ACC_W + BAND + c * 16, 16)])
    pltpu.sync_copy(stage_v, out_hbm.at[pl.ds(base_row, BAND)])


@functools.cache
def _make_sc_call(npad):
    return functools.partial(
        pl.kernel,
        mesh=plsc.VectorSubcoreMesh(core_axis_name="c", subcore_axis_name="s"),
        out_type=jax.ShapeDtypeStruct((H, W), jnp.float32),
        compiler_params=pltpu.CompilerParams(needs_layout_passes=False),
        scratch_types=[
            pltpu.VMEM((npad,), jnp.float32),
            pltpu.VMEM((npad,), jnp.float32),
            pltpu.VMEM((npad,), jnp.float32),
            pltpu.VMEM((npad,), jnp.float32),
            pltpu.VMEM((16,), jnp.float32),
            pltpu.VMEM((npad + 16,), jnp.int32),
            pltpu.VMEM((BAND * 16,), jnp.float32),
            pltpu.VMEM((BAND * ACC_W,), jnp.float32),
            pltpu.VMEM((BAND, W), jnp.float32),
        ],
    )(functools.partial(_sc_body, npad))


def _kernel_sc(pos_x, pos_y, height, width, background):
    n = pos_x.shape[0]
    npad = (n // 16 + 1) * 16
    pad = npad - n
    px = jnp.pad(pos_x, (0, pad))
    py = jnp.pad(pos_y, (0, pad), constant_values=1e6)
    h = jnp.pad(height, (0, pad))
    w = jnp.pad(width, (0, pad), constant_values=1.0)
    bg = jnp.full((16,), background, jnp.float32)
    return _make_sc_call(npad)(px, py, h, w, bg)


def kernel(x_grid, y_grid, pos_x, pos_y, height, width, background):
    n = pos_x.shape[0]
    n_tc = (n * 82 // 100) // BLK * BLK
    m = n - n_tc
    sc_img = _kernel_sc(pos_x[:m], pos_y[:m], height[:m], width[:m],
                        background)
    tc_img = _kernel_tc(pos_x[m:], pos_y[m:], height[m:], width[m:],
                        jnp.zeros((), jnp.float32))
    return sc_img + tc_img


# speedup vs baseline: 2.8230x; 1.5236x over previous
import functools

import jax
import jax.numpy as jnp
from jax import lax
from jax.experimental import pallas as pl
from jax.experimental.pallas import tpu as pltpu

H = 512
W = 512
WINDOW = 12

BLK = 2048


def _image_kernel(px_ref, py_ref, h_ref, w_ref, bg_ref, out_ref):
    k = pl.program_id(0)

    px = px_ref[...]
    py = py_ref[...]
    height = h_ref[...]
    width = w_ref[...]
    s = jnp.sqrt(0.5 * 1.4426950408889634) / width

    cols = lax.broadcasted_iota(jnp.int32, (BLK, W), 1).astype(jnp.float32)
    dx = cols * s[:, None] - (px * s)[:, None]
    fx = jnp.exp2(-(dx * dx))
    dy = cols * s[:, None] - (py * s)[:, None]
    fy = height[:, None] * jnp.exp2(-(dy * dy))

    acc = lax.dot_general(
        fy.astype(jnp.bfloat16), fx.astype(jnp.bfloat16),
        (((0,), (0,)), ((), ())),
        preferred_element_type=jnp.float32,
    )

    @pl.when(k == 0)
    def _():
        out_ref[...] = jnp.full((H, W), bg_ref[0, 0], jnp.float32)

    out_ref[...] += acc


def _kernel_tc(pos_x, pos_y, height, width, background):
    n = pos_x.shape[0]
    n_pad = ((n + BLK - 1) // BLK) * BLK
    pad = n_pad - n
    pos_x = jnp.pad(pos_x, (0, pad))
    pos_y = jnp.pad(pos_y, (0, pad))
    height = jnp.pad(height, (0, pad))
    width = jnp.pad(width, (0, pad), constant_values=1.0)
    bg = jnp.reshape(background, (1, 1)).astype(jnp.float32)

    grid = n_pad // BLK
    peaks_spec = pl.BlockSpec((BLK,), lambda k: (k,))
    return pl.pallas_call(
        _image_kernel,
        grid=(grid,),
        in_specs=[peaks_spec, peaks_spec, peaks_spec, peaks_spec,
                  pl.BlockSpec(memory_space=pltpu.SMEM)],
        out_specs=pl.BlockSpec((H, W), lambda k: (0, 0)),
        out_shape=jax.ShapeDtypeStruct((H, W), jnp.float32),
    )(pos_x, pos_y, height, width, bg)


from jax.experimental.pallas import tpu_sc as plsc

BAND = 16
NW = 32
NPAD = 10240
ACC_W = W + 2 * BAND
DWIN = 2 * WINDOW + 1


def _sc_body(npad, px_hbm, py_hbm, h_hbm, w_hbm, bg_hbm, out_hbm,
             px_v, py_v, h_v, w_v, bg_v, wl_v, hy_v, acc_v, stage_v):
    wid = lax.axis_index("s") * 2 + lax.axis_index("c")
    base_row = wid * BAND

    pltpu.sync_copy(px_hbm, px_v)
    pltpu.sync_copy(py_hbm, py_v)
    pltpu.sync_copy(h_hbm, h_v)
    pltpu.sync_copy(w_hbm, w_v)
    pltpu.sync_copy(bg_hbm, bg_v)

    iota = lax.iota(jnp.int32, 16)
    bgv = bg_v[...]

    dummy = jnp.full((16,), npad - 1, jnp.int32)

    def _wl_init(i, _):
        wl_v[pl.ds(i * 16, 16)] = dummy
        return 0

    lax.fori_loop(0, (npad + 16) // 16, _wl_init, 0)

    def _acc_init(i, _):
        acc_v[pl.ds(i * 16, 16)] = bgv
        return 0

    lax.fori_loop(0, (BAND * ACC_W) // 16, _acc_init, 0)

    lo = base_row - WINDOW
    hi = base_row + BAND - 1 + WINDOW

    def _scan(i, total):
        py16 = py_v[pl.ds(i * 16, 16)]
        ry = py16.astype(jnp.int32)
        m = (ry >= lo) & (ry <= hi)
        offs = total + plsc.cumsum(m.astype(jnp.int32)) - 1
        plsc.store_scatter(wl_v, [offs], iota + i * 16, mask=m)
        return total + plsc.all_reduce_population_count(m)

    zero_v = jnp.zeros((16,), jnp.int32)
    total_v = lax.fori_loop(0, npad // 16, _scan, zero_v)
    total = total_v[0]

    def _chunk(t, _):
        idx = wl_v[pl.ds(t * 16, 16)]
        pxv = plsc.load_gather(px_v, [idx])
        pyv = plsc.load_gather(py_v, [idx])
        hv = plsc.load_gather(h_v, [idx])
        wv = plsc.load_gather(w_v, [idx])
        ninv = -0.5 / (wv * wv)
        pxi = pxv.astype(jnp.int32)
        fx = pxv - pxi.astype(jnp.float32)
        colbase = pxi + (BAND - WINDOW)

        hx = []
        for d in range(DWIN):
            dd = (d - WINDOW) - fx
            hx.append(jnp.exp((dd * dd) * ninv))

        for r in range(BAND):
            dr = (base_row + r) - pyv
            hy_v[pl.ds(r * 16, 16)] = hv * jnp.exp((dr * dr) * ninv)

        def _row(e, _):
            rot = (iota + e) & 15
            hy_e = plsc.load_gather(hy_v, [rot * 16 + iota])
            base_addr = rot * ACC_W + colbase
            for d in range(DWIN):
                plsc.addupdate_scatter(acc_v, [base_addr + d], hy_e * hx[d])
            return 0

        lax.fori_loop(0, BAND, _row, 0)
        return 0

    lax.fori_loop(0, (total + 15) // 16, _chunk, 0)

    def _compact(r, _):
        for c in range(W // 16):
            stage_v[r, pl.ds(c * 16, 16)] = (
                acc_v[pl.ds(r * ACC_W + BAND + c * 16, 16)])
        return 0

    lax.fori_loop(0, BAND, _compact, 0)
    pltpu.sync_copy(stage_v, out_hbm.at[pl.ds(base_row, BAND)])


@functools.cache
def _make_sc_call(npad):
    return functools.partial(
        pl.kernel,
        mesh=plsc.VectorSubcoreMesh(core_axis_name="c", subcore_axis_name="s"),
        out_type=jax.ShapeDtypeStruct((H, W), jnp.float32),
        compiler_params=pltpu.CompilerParams(needs_layout_passes=False),
        scratch_types=[
            pltpu.VMEM((npad,), jnp.float32),
            pltpu.VMEM((npad,), jnp.float32),
            pltpu.VMEM((npad,), jnp.float32),
            pltpu.VMEM((npad,), jnp.float32),
            pltpu.VMEM((16,), jnp.float32),
            pltpu.VMEM((npad + 16,), jnp.int32),
            pltpu.VMEM((BAND * 16,), jnp.float32),
            pltpu.VMEM((BAND * ACC_W,), jnp.float32),
            pltpu.VMEM((BAND, W), jnp.float32),
        ],
    )(functools.partial(_sc_body, npad))


def _kernel_sc(pos_x, pos_y, height, width, background):
    n = pos_x.shape[0]
    npad = (n // 16 + 1) * 16
    pad = npad - n
    px = jnp.pad(pos_x, (0, pad))
    py = jnp.pad(pos_y, (0, pad), constant_values=1e6)
    h = jnp.pad(height, (0, pad))
    w = jnp.pad(width, (0, pad), constant_values=1.0)
    bg = jnp.full((16,), background, jnp.float32)
    return _make_sc_call(npad)(px, py, h, w, bg)


def kernel(x_grid, y_grid, pos_x, pos_y, height, width, background):
    n = pos_x.shape[0]
    n_tc = (n * 82 // 100) // BLK * BLK
    m = n - n_tc
    sc_img = _kernel_sc(pos_x[:m], pos_y[:m], height[:m], width[:m],
                        background)
    tc_img = _kernel_tc(pos_x[m:], pos_y[m:], height[m:], width[m:],
                        jnp.zeros((), jnp.float32))
    return sc_img + tc_img
